# baseline (speedup 1.0000x reference)
import jax
import jax.numpy as jnp
from jax.experimental import pallas as pl
from jax.experimental.pallas import tpu as pltpu


def _mish(h):
    sp = jnp.maximum(h, 0.0) + jnp.log1p(jnp.exp(-jnp.abs(h)))
    return h * jnp.tanh(sp)


def _se_kernel(x_ref, w1_ref, b1_ref, w2_ref, b2_ref, o_ref):
    xb = x_ref[0]
    m = jnp.mean(xb, axis=1, keepdims=True)
    h = jnp.dot(w1_ref[...], m,
                preferred_element_type=jnp.float32) + b1_ref[...]
    h = _mish(h)
    s = jnp.dot(w2_ref[...], h,
                preferred_element_type=jnp.float32) + b2_ref[...]
    s = jax.nn.sigmoid(s)
    o_ref[0] = xb * s


def kernel(x, W1, b1, W2, b2, *, interpret=False):
    B, C, H, W = x.shape
    HID = W1.shape[0]
    HW = H * W
    x2 = x.reshape(B, C, HW)
    b1c = b1.reshape(HID, 1)
    b2c = b2.reshape(C, 1)
    out = pl.pallas_call(
        _se_kernel,
        out_shape=jax.ShapeDtypeStruct((B, C, HW), x.dtype),
        grid=(B,),
        in_specs=[
            pl.BlockSpec((1, C, HW), lambda i: (i, 0, 0)),
            pl.BlockSpec((HID, C), lambda i: (0, 0)),
            pl.BlockSpec((HID, 1), lambda i: (0, 0)),
            pl.BlockSpec((C, HID), lambda i: (0, 0)),
            pl.BlockSpec((C, 1), lambda i: (0, 0)),
        ],
        out_specs=pl.BlockSpec((1, C, HW), lambda i: (i, 0, 0)),
        compiler_params=pltpu.CompilerParams(
            dimension_semantics=("parallel",),
        ),
        name="se_fused",
        interpret=interpret,
    )(x2, W1, b1c, W2, b2c)
    return out.reshape(B, C, H, W)
